# Initial kernel scaffold; baseline (speedup 1.0000x reference)
import jax
import jax.numpy as jnp
from jax.experimental import pallas as pl


def kernel(x, Wq, bq, Wk, bk, Wv, bv, W1, b1, W2, b2):
    raise NotImplementedError("write your pallas kernel here")


# speedup vs baseline: 1.3145x; 1.3145x over previous
import functools

import jax
import jax.numpy as jnp
import numpy as np
from jax.experimental import pallas as pl
from jax.experimental.pallas import tpu as pltpu

_KS = 25
_PAD = (_KS - 1) // 2
_TOPK = 8

_SA = 512
_SS = 256
_G = 4
_SC1 = 512
_SC2 = 256


def _decomp_qkv_body(xa, xb, wq, bq, wk, bk, wv, bv,
                     seas, tr1, q, k, v, *, sa, d):
    xe = jnp.concatenate([xa[0], xb[0, :2 * _PAD]], axis=0)
    trend = xe[0:sa]
    for o in range(1, _KS):
        trend = trend + xe[o:o + sa]
    trend = trend * (1.0 / _KS)
    s = xe[_PAD:_PAD + sa] - trend
    seas[0] = s
    tr1[0] = trend
    hi = jax.lax.Precision.HIGHEST
    q[0] = jax.lax.dot(s, wq[...], precision=hi) + bq[0]
    k[0] = jax.lax.dot(s, wk[...], precision=hi) + bk[0]
    v[0] = jax.lax.dot(s, wv[...], precision=hi) + bv[0]


def _score_body(i_ref, kb_ref, q_ref, kg_ref, lags_ref, scr, *,
                n, ng, p_total, t):
    p = pl.program_id(1)

    @pl.when(p == 0)
    def _zero():
        scr[...] = jnp.zeros_like(scr)

    i = i_ref[p]
    kb = kb_ref[p]
    jj = ng - 1 - kb
    qt = q_ref[0]
    kg = kg_ref[0]
    hi = jax.lax.Precision.HIGHEST
    a2 = jax.lax.dot_general(qt, kg, (((1,), (1,)), ((), ())),
                             precision=hi)
    rows = jax.lax.broadcasted_iota(jnp.int32, (_SS, 2 * _SS), 0)
    for g in range(_G):
        ag = a2[:, (_G - 1 - g) * _SS:(_G - g) * _SS]
        acc = jnp.concatenate(
            [ag, jnp.zeros((_SS, _SS), jnp.float32)], axis=1)
        for bb in range(8):
            bit = 1 << bb
            acc = jnp.where((rows & bit) != 0,
                            jnp.roll(acc, bit, axis=1), acc)
        c = jnp.sum(acc, axis=0, keepdims=True)
        e = jnp.roll(c, 1, axis=1)
        u = i - (jj * _G + g)
        riota = jax.lax.broadcasted_iota(jnp.int32, (n, 1), 0)
        upd = (jnp.where(riota == u - 1, 1.0, 0.0) * e[:, :_SS]
               + jnp.where(riota == u, 1.0, 0.0) * e[:, _SS:])
        scr[...] = scr[...] + upd

    @pl.when(p == p_total - 1)
    def _topk():
        s = scr[...]
        fi = (jax.lax.broadcasted_iota(jnp.int32, (n, _SS), 0) * _SS
              + jax.lax.broadcasted_iota(jnp.int32, (n, _SS), 1))
        s = jnp.where(fi == 0, -jnp.inf, s)
        lane = jax.lax.broadcasted_iota(jnp.int32, (1, 128), 1)
        lagvec = jnp.zeros((1, 128), jnp.int32)
        kk = min(_TOPK, t - 1)
        for q_i in range(kk):
            m = jnp.max(s)
            idx = jnp.min(jnp.where(s == m, fi, jnp.int32(2 ** 30)))
            lagvec = lagvec + jnp.where(lane == q_i, idx, 0)
            s = jnp.where(fi == idx, -jnp.inf, s)
        lags_ref[0] = lagvec


def _gather_body(lag_ref, vext_ref, ac_ref, vscr, sem, *, t, sc, d):
    b = pl.program_id(0)
    tt = pl.program_id(1)
    t0 = tt * sc
    kk = min(_TOPK, t - 1)

    @pl.when(tt == 0)
    def _load_v():
        cp = pltpu.make_async_copy(vext_ref.at[b], vscr, sem)
        cp.start()
        cp.wait()

    acc = jnp.zeros((sc, d), jnp.float32)
    for i in range(kk):
        lag = lag_ref[b, i]
        st = jax.lax.rem(t0 - lag + t, t)
        stq = st // 8
        r = st - stq * 8
        g1 = vscr[pl.ds(stq, sc // 8)].reshape(sc, d)
        gx = vscr[pl.ds(stq + sc // 8, 1)].reshape(8, d)

        def _mk(rr):
            def _br():
                if rr == 0:
                    return g1
                return jnp.concatenate([g1[rr:], gx[:rr]], axis=0)
            return _br

        acc = acc + jax.lax.switch(r, [_mk(rr) for rr in range(8)])
    ac_ref[0] = acc * (1.0 / kk)


def _ffn_body(sa_ref, sb_ref, aa_ref, ap_ref, an_ref, tr_ref,
              w1, b1, w2, b2, outs, outt, *, t, sc, d):
    tt = pl.program_id(1)
    t0 = tt * sc
    ext_s = jnp.concatenate([sa_ref[0], sb_ref[0, :2 * _PAD]], axis=0)
    ext_a = jnp.concatenate(
        [ap_ref[0, 32 - _PAD:32], aa_ref[0], an_ref[0, :_PAD]], axis=0)
    s2 = ext_s + ext_a
    gr = jax.lax.broadcasted_iota(jnp.int32, (sc + 2 * _PAD, 1), 0) \
        + (t0 - _PAD)
    s2 = jnp.where(gr < 0, s2[_PAD:_PAD + 1],
                   jnp.where(gr > t - 1, s2[sc + _PAD - 1:sc + _PAD], s2))
    trend2 = s2[0:sc]
    for o in range(1, _KS):
        trend2 = trend2 + s2[o:o + sc]
    trend2 = trend2 * (1.0 / _KS)
    seas2 = s2[_PAD:_PAD + sc] - trend2
    hi = jax.lax.Precision.HIGHEST
    h = jax.lax.dot(seas2, w1[...], precision=hi) + b1[0]
    h = h * 0.5 * (1.0 + jax.lax.erf(h * np.float32(1.0 / np.sqrt(2.0))))
    ff = jax.lax.dot(h, w2[...], precision=hi) + b2[0]
    outs[0] = seas2 + ff
    outt[0] = tr_ref[0] + trend2


def kernel(x, Wq, bq, Wk, bk, Wv, bv, W1, b1, W2, b2):
    B, T, D = x.shape
    F = W1.shape[1]
    f32 = jnp.float32

    xpad = jnp.pad(x, ((0, 0), (_PAD, _PAD), (0, 0)), mode='edge')
    nta = T // _SA
    bd_spec = pl.BlockSpec((1, D), lambda b, tt: (0, 0))
    w_spec = pl.BlockSpec((D, D), lambda b, tt: (0, 0))
    tile_a = pl.BlockSpec((1, _SA, D), lambda b, tt: (b, tt, 0))
    seas, tr1, q, k, v = pl.pallas_call(
        functools.partial(_decomp_qkv_body, sa=_SA, d=D),
        grid=(B, nta),
        in_specs=[
            tile_a,
            pl.BlockSpec((1, 32, D), lambda b, tt: (b, (tt + 1) * (_SA // 32), 0)),
            w_spec, bd_spec, w_spec, bd_spec, w_spec, bd_spec,
        ],
        out_specs=[tile_a] * 5,
        out_shape=[jax.ShapeDtypeStruct((B, T, D), f32)] * 5,
    )(xpad, xpad, Wq, bq.reshape(1, D), Wk, bk.reshape(1, D),
      Wv, bv.reshape(1, D))

    n = T // _SS
    ng = n // _G
    i_list, kb_list = [], []
    for jj in range(ng):
        for i in range(jj * _G, n):
            i_list.append(i)
            kb_list.append(ng - 1 - jj)
    p_total = len(i_list)
    i_arr = jnp.asarray(np.array(i_list, np.int32))
    kb_arr = jnp.asarray(np.array(kb_list, np.int32))
    krev = jnp.flip(k, axis=1)

    grid_b = pltpu.PrefetchScalarGridSpec(
        num_scalar_prefetch=2,
        grid=(B, p_total),
        in_specs=[
            pl.BlockSpec((1, _SS, D), lambda b, p, ia, kb: (b, ia[p], 0)),
            pl.BlockSpec((1, _G * _SS, D), lambda b, p, ia, kb: (b, kb[p], 0)),
        ],
        out_specs=pl.BlockSpec((1, 1, 128), lambda b, p, ia, kb: (b, 0, 0)),
        scratch_shapes=[pltpu.VMEM((n, _SS), f32)],
    )
    lags_out = pl.pallas_call(
        functools.partial(_score_body, n=n, ng=ng, p_total=p_total, t=T),
        grid_spec=grid_b,
        out_shape=jax.ShapeDtypeStruct((B, 1, 128), jnp.int32),
    )(i_arr, kb_arr, q, krev)
    lags = lags_out[:, 0, :_TOPK]

    v_ext = jnp.concatenate([v, v[:, :_SC1 + 8]], axis=1)
    v_ext = v_ext.reshape(B, (T + _SC1 + 8) // 8, 8, D)
    grid_c1 = pltpu.PrefetchScalarGridSpec(
        num_scalar_prefetch=1,
        grid=(B, T // _SC1),
        in_specs=[
            pl.BlockSpec(memory_space=pl.ANY),
        ],
        out_specs=pl.BlockSpec((1, _SC1, D), lambda b, tt, lg: (b, tt, 0)),
        scratch_shapes=[
            pltpu.VMEM(v_ext.shape[1:], jnp.float32),
            pltpu.SemaphoreType.DMA,
        ],
    )
    ac = pl.pallas_call(
        functools.partial(_gather_body, t=T, sc=_SC1, d=D),
        grid_spec=grid_c1,
        out_shape=jax.ShapeDtypeStruct((B, T, D), f32),
    )(lags, v_ext)

    spad = jnp.pad(seas, ((0, 0), (_PAD, _PAD), (0, 0)))
    nt2 = T // _SC2
    nb32 = T // 32
    tile_c = pl.BlockSpec((1, _SC2, D), lambda b, tt: (b, tt, 0))
    outs, outt = pl.pallas_call(
        functools.partial(_ffn_body, t=T, sc=_SC2, d=D),
        grid=(B, nt2),
        in_specs=[
            tile_c,
            pl.BlockSpec((1, 32, D),
                         lambda b, tt: (b, (tt + 1) * (_SC2 // 32), 0)),
            tile_c,
            pl.BlockSpec((1, 32, D),
                         lambda b, tt: (b, jnp.maximum(tt * (_SC2 // 32) - 1, 0), 0)),
            pl.BlockSpec((1, 32, D),
                         lambda b, tt: (b, jnp.minimum((tt + 1) * (_SC2 // 32), nb32 - 1), 0)),
            tile_c,
            pl.BlockSpec((D, F), lambda b, tt: (0, 0)),
            pl.BlockSpec((1, F), lambda b, tt: (0, 0)),
            pl.BlockSpec((F, D), lambda b, tt: (0, 0)),
            pl.BlockSpec((1, D), lambda b, tt: (0, 0)),
        ],
        out_specs=[tile_c, tile_c],
        out_shape=[jax.ShapeDtypeStruct((B, T, D), f32)] * 2,
    )(spad, spad, ac, ac, ac, tr1,
      W1, b1.reshape(1, F), W2, b2.reshape(1, D))

    return (outs, outt)
